# Initial kernel scaffold; baseline (speedup 1.0000x reference)
import jax
import jax.numpy as jnp
from jax.experimental import pallas as pl


def kernel(img, chargrid_map, embedding_table):
    raise NotImplementedError("write your pallas kernel here")


# speedup vs baseline: 2.3415x; 2.3415x over previous
import functools

import jax
import jax.numpy as jnp
from jax import lax
from jax.experimental import pallas as pl
from jax.experimental.pallas import tpu as pltpu
from jax.experimental.pallas import tpu_sc as plsc

_NC = 2
_NS = 16
_NW = _NC * _NS

_IB = 128
_CHUNK = 512
_K = _CHUNK // _IB


def _make_sc_gather(n_rows: int, d: int, vocab: int):
    per_w = n_rows // _NW
    n_chunks = per_w // _CHUNK
    mesh = plsc.VectorSubcoreMesh(core_axis_name="c", subcore_axis_name="s")

    @functools.partial(
        pl.kernel,
        mesh=mesh,
        compiler_params=pltpu.CompilerParams(use_tc_tiling_on_sc=False),
        out_type=jax.ShapeDtypeStruct((n_rows, d), jnp.float32),
        scratch_types=[
            pltpu.VMEM((n_chunks, _K, _IB), jnp.int32),
            pltpu.VMEM((_CHUNK, d), jnp.float32),
            pltpu.SemaphoreType.DMA,
        ],
    )
    def sc_gather(table_hbm, idx_hbm, out_hbm, idx_v, rows_v, sem):
        wid = lax.axis_index("s") * _NC + lax.axis_index("c")
        pltpu.sync_copy(idx_hbm.at[wid], idx_v)

        def body(c, carry):
            cps = [
                pltpu.async_copy(
                    table_hbm.at[idx_v.at[c, j]],
                    rows_v.at[pl.ds(j * _IB, _IB)],
                    sem,
                )
                for j in range(_K)
            ]
            for cp in cps:
                cp.wait()
            pltpu.sync_copy(
                rows_v, out_hbm.at[pl.ds(wid * per_w + c * _CHUNK, _CHUNK)]
            )
            return carry

        lax.fori_loop(0, n_chunks, body, 0)

    return sc_gather


def _tc_transpose(g3, tile: int = 1024):
    b, hw, d = g3.shape

    def body(i_ref, o_ref):
        o_ref[...] = jnp.swapaxes(i_ref[...], 1, 2)

    return pl.pallas_call(
        body,
        grid=(b, hw // tile),
        in_specs=[pl.BlockSpec((1, tile, d), lambda bi, ti: (bi, ti, 0))],
        out_specs=pl.BlockSpec((1, d, tile), lambda bi, ti: (bi, 0, ti)),
        out_shape=jax.ShapeDtypeStruct((b, d, hw), jnp.float32),
    )(g3)


def kernel(img, chargrid_map, embedding_table):
    b, _, h, w = chargrid_map.shape
    vocab, d = embedding_table.shape
    n = b * h * w

    idx = chargrid_map.reshape(_NW, (n // _NW) // _CHUNK, _K, _IB)
    gathered = _make_sc_gather(n, d, vocab)(embedding_table, idx)
    g3 = gathered.reshape(b, h * w, d)
    out = _tc_transpose(g3)
    return out.reshape(b, d, h, w)


# speedup vs baseline: 3.6647x; 1.5651x over previous
import functools

import jax
import jax.numpy as jnp
from jax import lax
from jax.experimental import pallas as pl
from jax.experimental.pallas import tpu as pltpu
from jax.experimental.pallas import tpu_sc as plsc

_NC = 2
_NS = 16
_NW = _NC * _NS

_P = 8192


def _tc_transpose_table(table, vocab_pad):
    vocab, d = table.shape
    tile = 512

    def body(i_ref, o_ref):
        o_ref[...] = i_ref[...].T

    return pl.pallas_call(
        body,
        grid=(vocab_pad // tile,),
        in_specs=[pl.BlockSpec((tile, d), lambda t: (t, 0))],
        out_specs=pl.BlockSpec((d, tile), lambda t: (0, t)),
        out_shape=jax.ShapeDtypeStruct((d, vocab_pad), jnp.float32),
    )(table)


def _make_sc_lookup(n_pix: int, d: int, vocab_pad: int, hw: int):
    n_chunks = n_pix // _P
    chunks_per_img = hw // _P
    mesh = plsc.VectorSubcoreMesh(core_axis_name="c", subcore_axis_name="s")

    @functools.partial(
        pl.kernel,
        mesh=mesh,
        compiler_params=pltpu.CompilerParams(
            use_tc_tiling_on_sc=False, needs_layout_passes=False),
        out_type=jax.ShapeDtypeStruct((n_pix // hw, d, hw), jnp.float32),
        scratch_types=[
            pltpu.VMEM((vocab_pad,), jnp.float32),
            pltpu.VMEM((vocab_pad,), jnp.float32),
            pltpu.VMEM((_P,), jnp.int32),
            pltpu.VMEM((_P,), jnp.int32),
            pltpu.VMEM((_P,), jnp.float32),
            pltpu.VMEM((_P,), jnp.float32),
            pltpu.VMEM((_P,), jnp.float32),
            pltpu.VMEM((_P,), jnp.float32),
            pltpu.SemaphoreType.DMA,
            pltpu.SemaphoreType.DMA,
            pltpu.SemaphoreType.DMA,
            pltpu.SemaphoreType.DMA,
        ],
    )
    def sc_lookup(tableT_hbm, idx_hbm, out_hbm, t0_v, t1_v,
                  idx0_v, idx1_v, o00_v, o10_v, o01_v, o11_v,
                  isem0, isem1, osem0, osem1):
        wid = lax.axis_index("s") * _NC + lax.axis_index("c")
        d0 = wid * 2
        d1 = wid * 2 + 1
        pltpu.sync_copy(tableT_hbm.at[d0], t0_v)
        pltpu.sync_copy(tableT_hbm.at[d1], t1_v)

        idx_bufs = (idx0_v, idx1_v)
        out_bufs = ((o00_v, o10_v), (o01_v, o11_v))
        isems = (isem0, isem1)
        osems = (osem0, osem1)

        def start_idx(c):
            s = c % 2
            return pltpu.async_copy(
                idx_hbm.at[pl.ds(c * _P, _P)], idx_bufs[s], isems[s])

        def compute(s):
            ob0, ob1 = out_bufs[s]
            ib = idx_bufs[s]

            def gbody(i, carry):
                base = i * 16
                vi = ib[pl.ds(base, 16)]
                ob0[pl.ds(base, 16)] = plsc.load_gather(t0_v, [vi])
                ob1[pl.ds(base, 16)] = plsc.load_gather(t1_v, [vi])
                return carry

            lax.fori_loop(0, _P // 16, gbody, 0)

        def start_out(c):
            s = c % 2
            b, p = divmod(c, chunks_per_img)
            ob0, ob1 = out_bufs[s]
            return (
                pltpu.async_copy(
                    ob0, out_hbm.at[b, d0, pl.ds(p * _P, _P)], osems[s]),
                pltpu.async_copy(
                    ob1, out_hbm.at[b, d1, pl.ds(p * _P, _P)], osems[s]),
            )

        pending_idx = [start_idx(0)]
        pending_out = [None, None]
        for c in range(n_chunks):
            s = c % 2
            if c + 1 < n_chunks:
                pending_idx.append(start_idx(c + 1))
            if pending_out[s] is not None:
                for cp in pending_out[s]:
                    cp.wait()
            pending_idx[c].wait()
            compute(s)
            pending_out[s] = start_out(c)
        for pair in pending_out:
            if pair is not None:
                for cp in pair:
                    cp.wait()

    return sc_lookup


def kernel(img, chargrid_map, embedding_table):
    b, _, h, w = chargrid_map.shape
    vocab, d = embedding_table.shape
    hw = h * w
    n = b * hw
    vocab_pad = -(-vocab // 512) * 512

    table_t = _tc_transpose_table(embedding_table, vocab_pad)
    idx = chargrid_map.reshape(n)
    out = _make_sc_lookup(n, d, vocab_pad, hw)(table_t, idx)
    return out.reshape(b, d, h, w)


# speedup vs baseline: 5.6938x; 1.5537x over previous
import functools

import jax
import jax.numpy as jnp
from jax import lax
from jax.experimental import pallas as pl
from jax.experimental.pallas import tpu as pltpu
from jax.experimental.pallas import tpu_sc as plsc

_NC = 2
_NS = 16
_NW = _NC * _NS

_P = 8192


def _tc_transpose_table(table, vocab_pad):
    vocab, d = table.shape
    tile = 512

    def body(i_ref, o_ref):
        o_ref[...] = i_ref[...].T

    return pl.pallas_call(
        body,
        grid=(vocab_pad // tile,),
        in_specs=[pl.BlockSpec((tile, d), lambda t: (t, 0))],
        out_specs=pl.BlockSpec((d, tile), lambda t: (0, t)),
        out_shape=jax.ShapeDtypeStruct((d, vocab_pad), jnp.float32),
    )(table)


def _make_sc_lookup(n_pix: int, d: int, vocab_pad: int, hw: int):
    n_chunks = n_pix // _P
    chunks_per_img = hw // _P
    mesh = plsc.VectorSubcoreMesh(core_axis_name="c", subcore_axis_name="s")

    @functools.partial(
        pl.kernel,
        mesh=mesh,
        compiler_params=pltpu.CompilerParams(
            use_tc_tiling_on_sc=False, needs_layout_passes=False),
        out_type=jax.ShapeDtypeStruct((n_pix // hw, d, hw), jnp.float32),
        scratch_types=[
            pltpu.VMEM((vocab_pad,), jnp.float32),
            pltpu.VMEM((vocab_pad,), jnp.float32),
            pltpu.VMEM((_P,), jnp.int32),
            pltpu.VMEM((_P,), jnp.int32),
            pltpu.VMEM((_P,), jnp.float32),
            pltpu.VMEM((_P,), jnp.float32),
            pltpu.VMEM((_P,), jnp.float32),
            pltpu.VMEM((_P,), jnp.float32),
            pltpu.SemaphoreType.DMA,
            pltpu.SemaphoreType.DMA,
            pltpu.SemaphoreType.DMA,
            pltpu.SemaphoreType.DMA,
        ],
    )
    def sc_lookup(tableT_hbm, idx_hbm, out_hbm, t0_v, t1_v,
                  idx0_v, idx1_v, o00_v, o10_v, o01_v, o11_v,
                  isem0, isem1, osem0, osem1):
        wid = lax.axis_index("s") * _NC + lax.axis_index("c")
        d0 = wid * 2
        d1 = wid * 2 + 1
        pltpu.sync_copy(tableT_hbm.at[d0], t0_v)
        pltpu.sync_copy(tableT_hbm.at[d1], t1_v)

        idx_bufs = (idx0_v, idx1_v)
        out_bufs = ((o00_v, o10_v), (o01_v, o11_v))
        isems = (isem0, isem1)
        osems = (osem0, osem1)

        def start_idx(c):
            s = c % 2
            return pltpu.async_copy(
                idx_hbm.at[pl.ds(c * _P, _P)], idx_bufs[s], isems[s])

        def compute(s):
            ob0, ob1 = out_bufs[s]
            ib = idx_bufs[s]

            @plsc.parallel_loop(0, _P, step=16, unroll=8)
            def gbody(i):
                vi = ib[pl.ds(i, 16)]
                ob0[pl.ds(i, 16)] = plsc.load_gather(t0_v, [vi])
                ob1[pl.ds(i, 16)] = plsc.load_gather(t1_v, [vi])

        def start_out(c):
            s = c % 2
            b, p = divmod(c, chunks_per_img)
            ob0, ob1 = out_bufs[s]
            return (
                pltpu.async_copy(
                    ob0, out_hbm.at[b, d0, pl.ds(p * _P, _P)], osems[s]),
                pltpu.async_copy(
                    ob1, out_hbm.at[b, d1, pl.ds(p * _P, _P)], osems[s]),
            )

        pending_idx = [start_idx(0)]
        pending_out = [None, None]
        for c in range(n_chunks):
            s = c % 2
            if c + 1 < n_chunks:
                pending_idx.append(start_idx(c + 1))
            if pending_out[s] is not None:
                for cp in pending_out[s]:
                    cp.wait()
            pending_idx[c].wait()
            compute(s)
            pending_out[s] = start_out(c)
        for pair in pending_out:
            if pair is not None:
                for cp in pair:
                    cp.wait()

    return sc_lookup


def kernel(img, chargrid_map, embedding_table):
    b, _, h, w = chargrid_map.shape
    vocab, d = embedding_table.shape
    hw = h * w
    n = b * hw
    vocab_pad = -(-vocab // 512) * 512

    table_t = _tc_transpose_table(embedding_table, vocab_pad)
    idx = chargrid_map.reshape(n)
    out = _make_sc_lookup(n, d, vocab_pad, hw)(table_t, idx)
    return out.reshape(b, d, h, w)


# speedup vs baseline: 10.2780x; 1.8051x over previous
import functools

import jax
import jax.numpy as jnp
from jax import lax
from jax.experimental import pallas as pl
from jax.experimental.pallas import tpu as pltpu
from jax.experimental.pallas import tpu_sc as plsc

_NC = 2
_NS = 16
_NW = _NC * _NS

_TH = 8
_TW = 128


def _tc_transpose_table(table, vocab_pad):
    vocab, d = table.shape
    tile = 2048

    def body(i_ref, o_ref):
        o_ref[...] = i_ref[...].T

    return pl.pallas_call(
        body,
        grid=(vocab_pad // tile,),
        in_specs=[pl.BlockSpec((tile, d), lambda t: (t, 0))],
        out_specs=pl.BlockSpec((d, tile), lambda t: (0, t)),
        out_shape=jax.ShapeDtypeStruct((d, vocab_pad), jnp.float32),
    )(table)


def _make_sc_lookup(b: int, d: int, h: int, w: int, vocab_pad: int):
    nbh = h // _TH
    nbw = w // _TW
    px = nbw * _TH * _TW
    n_chunks = b * nbh
    mesh = plsc.VectorSubcoreMesh(core_axis_name="c", subcore_axis_name="s")

    @functools.partial(
        pl.kernel,
        mesh=mesh,
        compiler_params=pltpu.CompilerParams(
            use_tc_tiling_on_sc=True, needs_layout_passes=False),
        out_type=jax.ShapeDtypeStruct((b, d, h, w), jnp.float32),
        scratch_types=[
            pltpu.VMEM((vocab_pad,), jnp.float32),
            pltpu.VMEM((vocab_pad,), jnp.float32),
            pltpu.VMEM((2, nbw * _TH, _TW), jnp.int32),
            pltpu.VMEM((2, nbw * _TH, _TW), jnp.float32),
            pltpu.VMEM((2, nbw * _TH, _TW), jnp.float32),
            pltpu.SemaphoreType.DMA,
            pltpu.SemaphoreType.DMA,
            pltpu.SemaphoreType.DMA,
            pltpu.SemaphoreType.DMA,
        ],
    )
    def sc_lookup(tableT_hbm, idx_hbm, out_hbm, t0_v, t1_v,
                  idx_v, o0_v, o1_v, isem0, isem1, osem0, osem1):
        wid = lax.axis_index("s") * _NC + lax.axis_index("c")
        d0 = wid * 2
        d1 = wid * 2 + 1
        pltpu.sync_copy(tableT_hbm.at[d0], t0_v)
        pltpu.sync_copy(tableT_hbm.at[d1], t1_v)

        isems = (isem0, isem1)
        osems = (osem0, osem1)

        def coords(c):
            bi = c // nbh
            hb = c % nbh
            return bi, hb * _TH

        def start_idx(c, s):
            bi, h0 = coords(c)
            for wb in range(nbw):
                pltpu.async_copy(
                    idx_hbm.at[bi, pl.ds(h0, _TH), pl.ds(wb * _TW, _TW)],
                    idx_v.at[s, pl.ds(wb * _TH, _TH)],
                    isems[s],
                )

        def wait_idx(s):
            pltpu.make_async_copy(
                idx_hbm.at[0, pl.ds(0, nbw * _TH), pl.ds(0, _TW)],
                idx_v.at[s],
                isems[s],
            ).wait()

        def start_out(c, s):
            bi, h0 = coords(c)
            for dd, ob in ((d0, o0_v), (d1, o1_v)):
                for wb in range(nbw):
                    pltpu.async_copy(
                        ob.at[s, pl.ds(wb * _TH, _TH)],
                        out_hbm.at[bi, dd, pl.ds(h0, _TH),
                                   pl.ds(wb * _TW, _TW)],
                        osems[s],
                    )

        def wait_out(s):
            for ob in (o0_v, o1_v):
                pltpu.make_async_copy(
                    ob.at[s],
                    out_hbm.at[0, 0, pl.ds(0, nbw * _TH), pl.ds(0, _TW)],
                    osems[s],
                ).wait()

        def compute(s):
            @plsc.parallel_loop(0, nbw * _TH * _TW, step=16, unroll=8)
            def gbody(i):
                row = i >> 7
                col = i & (_TW - 1)
                vi = idx_v[s, row, pl.ds(col, 16)]
                o0_v[s, row, pl.ds(col, 16)] = plsc.load_gather(t0_v, [vi])
                o1_v[s, row, pl.ds(col, 16)] = plsc.load_gather(t1_v, [vi])

        start_idx(0, 0)
        start_idx(1, 1)

        def body(j, carry):
            ca = 2 * j
            cb = 2 * j + 1

            @pl.when(j >= 1)
            def _():
                wait_out(0)

            wait_idx(0)
            compute(0)
            start_out(ca, 0)

            @pl.when(ca + 2 < n_chunks)
            def _():
                start_idx(ca + 2, 0)

            @pl.when(j >= 1)
            def _():
                wait_out(1)

            wait_idx(1)
            compute(1)
            start_out(cb, 1)

            @pl.when(cb + 2 < n_chunks)
            def _():
                start_idx(cb + 2, 1)

            return carry

        lax.fori_loop(0, n_chunks // 2, body, 0)
        wait_out(0)
        wait_out(1)

    return sc_lookup


def kernel(img, chargrid_map, embedding_table):
    b, _, h, w = chargrid_map.shape
    vocab, d = embedding_table.shape
    vocab_pad = -(-vocab // 2048) * 2048

    table_t = _tc_transpose_table(embedding_table, vocab_pad)
    idx = chargrid_map.reshape(b, h, w)
    return _make_sc_lookup(b, d, h, w, vocab_pad)(table_t, idx)


# speedup vs baseline: 11.6139x; 1.1300x over previous
import functools

import jax
import jax.numpy as jnp
from jax import lax
from jax.experimental import pallas as pl
from jax.experimental.pallas import tpu as pltpu
from jax.experimental.pallas import tpu_sc as plsc

_NC = 2
_NS = 16
_NW = _NC * _NS

_TH = 8
_TW = 128


def _tc_transpose_table(table, vocab_pad):
    vocab, d = table.shape
    tile = 2048

    def body(i_ref, o_ref):
        o_ref[...] = i_ref[...].T

    return pl.pallas_call(
        body,
        grid=(vocab_pad // tile,),
        in_specs=[pl.BlockSpec((tile, d), lambda t: (t, 0))],
        out_specs=pl.BlockSpec((d, tile), lambda t: (0, t)),
        out_shape=jax.ShapeDtypeStruct((d, vocab_pad), jnp.float32),
    )(table)


def _make_sc_lookup(b: int, d: int, h: int, w: int, vocab_pad: int):
    _CH = 2
    nbh = h // (_TH * _CH)
    nbw = w // _TW
    rows = _CH * _TH
    n_chunks = b * nbh
    mesh = plsc.VectorSubcoreMesh(core_axis_name="c", subcore_axis_name="s")

    @functools.partial(
        pl.kernel,
        mesh=mesh,
        compiler_params=pltpu.CompilerParams(
            use_tc_tiling_on_sc=True, needs_layout_passes=False),
        out_type=jax.ShapeDtypeStruct((b, d, h, w), jnp.float32),
        scratch_types=[
            pltpu.VMEM((vocab_pad,), jnp.float32),
            pltpu.VMEM((vocab_pad,), jnp.float32),
            pltpu.VMEM((2, nbw * rows, _TW), jnp.int32),
            pltpu.VMEM((2, nbw * rows, _TW), jnp.float32),
            pltpu.VMEM((2, nbw * rows, _TW), jnp.float32),
            pltpu.SemaphoreType.DMA,
            pltpu.SemaphoreType.DMA,
            pltpu.SemaphoreType.DMA,
            pltpu.SemaphoreType.DMA,
        ],
    )
    def sc_lookup(tableT_hbm, idx_hbm, out_hbm, t0_v, t1_v,
                  idx_v, o0_v, o1_v, isem0, isem1, osem0, osem1):
        wid = lax.axis_index("s") * _NC + lax.axis_index("c")
        d0 = wid * 2
        d1 = wid * 2 + 1

        isems = (isem0, isem1)
        osems = (osem0, osem1)

        def coords(c):
            bi = c // nbh
            hb = c % nbh
            return bi, hb * rows

        def start_idx(c, s):
            bi, h0 = coords(c)
            for wb in range(nbw):
                pltpu.async_copy(
                    idx_hbm.at[bi, 0, pl.ds(h0, rows), pl.ds(wb * _TW, _TW)],
                    idx_v.at[s, pl.ds(wb * rows, rows)],
                    isems[s],
                )

        def wait_idx(s):
            pltpu.make_async_copy(
                idx_hbm.at[0, 0, pl.ds(0, nbw * rows), pl.ds(0, _TW)],
                idx_v.at[s],
                isems[s],
            ).wait()

        def start_out(c, s):
            bi, h0 = coords(c)
            for dd, ob in ((d0, o0_v), (d1, o1_v)):
                for wb in range(nbw):
                    pltpu.async_copy(
                        ob.at[s, pl.ds(wb * rows, rows)],
                        out_hbm.at[bi, dd, pl.ds(h0, rows),
                                   pl.ds(wb * _TW, _TW)],
                        osems[s],
                    )

        def wait_out(s):
            for ob in (o0_v, o1_v):
                pltpu.make_async_copy(
                    ob.at[s],
                    out_hbm.at[0, 0, pl.ds(0, nbw * rows), pl.ds(0, _TW)],
                    osems[s],
                ).wait()

        def compute(s):
            @plsc.parallel_loop(0, nbw * rows * _TW, step=16, unroll=8)
            def gbody(i):
                row = i >> 7
                col = i & (_TW - 1)
                vi = idx_v[s, row, pl.ds(col, 16)]
                o0_v[s, row, pl.ds(col, 16)] = plsc.load_gather(t0_v, [vi])
                o1_v[s, row, pl.ds(col, 16)] = plsc.load_gather(t1_v, [vi])

        start_idx(0, 0)
        start_idx(1, 1)
        pltpu.sync_copy(tableT_hbm.at[d0], t0_v)
        pltpu.sync_copy(tableT_hbm.at[d1], t1_v)

        def body(j, carry):
            ca = 2 * j
            cb = 2 * j + 1

            @pl.when(j >= 1)
            def _():
                wait_out(0)

            wait_idx(0)
            compute(0)
            start_out(ca, 0)

            @pl.when(ca + 2 < n_chunks)
            def _():
                start_idx(ca + 2, 0)

            @pl.when(j >= 1)
            def _():
                wait_out(1)

            wait_idx(1)
            compute(1)
            start_out(cb, 1)

            @pl.when(cb + 2 < n_chunks)
            def _():
                start_idx(cb + 2, 1)

            return carry

        lax.fori_loop(0, n_chunks // 2, body, 0)
        wait_out(0)
        wait_out(1)

    return sc_lookup


def kernel(img, chargrid_map, embedding_table):
    b, _, h, w = chargrid_map.shape
    vocab, d = embedding_table.shape
    vocab_pad = -(-vocab // 2048) * 2048

    table_t = _tc_transpose_table(embedding_table, vocab_pad)
    return _make_sc_lookup(b, d, h, w, vocab_pad)(table_t, chargrid_map)


# speedup vs baseline: 15.6909x; 1.3510x over previous
import functools

import jax
import jax.numpy as jnp
from jax import lax
from jax.experimental import pallas as pl
from jax.experimental.pallas import tpu as pltpu
from jax.experimental.pallas import tpu_sc as plsc

_NC = 2
_NS = 16
_NW = _NC * _NS

_TH = 8
_TW = 128


def _tc_transpose_table(table, vocab_pad):
    vocab, d = table.shape
    tile = 2048

    def body(i_ref, o_ref):
        o_ref[...] = i_ref[...].T

    return pl.pallas_call(
        body,
        grid=(vocab_pad // tile,),
        in_specs=[pl.BlockSpec((tile, d), lambda t: (t, 0))],
        out_specs=pl.BlockSpec((d, tile), lambda t: (0, t)),
        out_shape=jax.ShapeDtypeStruct((d, vocab_pad), jnp.float32),
    )(table)


def _make_sc_lookup(b: int, d: int, h: int, w: int, vocab_pad: int):
    _CH = 2
    nbh = h // (_TH * _CH)
    nbw = w // _TW
    rows = _CH * _TH
    n_chunks = b * nbh
    mesh = plsc.VectorSubcoreMesh(core_axis_name="c", subcore_axis_name="s")

    @functools.partial(
        pl.kernel,
        mesh=mesh,
        compiler_params=pltpu.CompilerParams(
            use_tc_tiling_on_sc=True, needs_layout_passes=False),
        out_type=jax.ShapeDtypeStruct((b, d, h, w), jnp.float32),
        scratch_types=[
            pltpu.VMEM((vocab_pad,), jnp.float32),
            pltpu.VMEM((vocab_pad,), jnp.float32),
            pltpu.VMEM((2, nbw * rows, _TW), jnp.int32),
            pltpu.VMEM((2, nbw * rows, _TW), jnp.float32),
            pltpu.VMEM((2, nbw * rows, _TW), jnp.float32),
            pltpu.VMEM_SHARED((2, h, w), jnp.int32),
            pltpu.SemaphoreType.DMA,
            pltpu.SemaphoreType.DMA,
            pltpu.SemaphoreType.DMA,
            pltpu.SemaphoreType.DMA,
            pltpu.SemaphoreType.DMA,
        ],
    )
    def sc_lookup(tableT_hbm, idx_hbm, out_hbm, t0_v, t1_v,
                  idx_v, o0_v, o1_v, idx_sp, isem0, isem1, osem0, osem1,
                  ssem):
        wid = lax.axis_index("s") * _NC + lax.axis_index("c")
        sid = lax.axis_index("s")
        d0 = wid * 2
        d1 = wid * 2 + 1

        isems = (isem0, isem1)
        osems = (osem0, osem1)

        rows_per_sid = h // _NS

        def start_staging(im):
            pltpu.async_copy(
                idx_hbm.at[im, 0, pl.ds(sid * rows_per_sid, rows_per_sid)],
                idx_sp.at[im % 2, pl.ds(sid * rows_per_sid, rows_per_sid)],
                ssem,
            )

        def finish_staging():
            pltpu.make_async_copy(
                idx_hbm.at[0, 0, pl.ds(0, rows_per_sid)],
                idx_sp.at[0, pl.ds(0, rows_per_sid)],
                ssem,
            ).wait()
            plsc.subcore_barrier()

        def coords(c):
            bi = c // nbh
            hb = c % nbh
            return bi, hb * rows

        def start_idx(c, s):
            bi, h0 = coords(c)
            for wb in range(nbw):
                pltpu.async_copy(
                    idx_sp.at[bi % 2, pl.ds(h0, rows), pl.ds(wb * _TW, _TW)],
                    idx_v.at[s, pl.ds(wb * rows, rows)],
                    isems[s],
                )

        def wait_idx(s):
            pltpu.make_async_copy(
                idx_sp.at[0, pl.ds(0, nbw * rows), pl.ds(0, _TW)],
                idx_v.at[s],
                isems[s],
            ).wait()

        def start_out(c, s):
            bi, h0 = coords(c)
            for dd, ob in ((d0, o0_v), (d1, o1_v)):
                for wb in range(nbw):
                    pltpu.async_copy(
                        ob.at[s, pl.ds(wb * rows, rows)],
                        out_hbm.at[bi, dd, pl.ds(h0, rows),
                                   pl.ds(wb * _TW, _TW)],
                        osems[s],
                    )

        def wait_out(s):
            for ob in (o0_v, o1_v):
                pltpu.make_async_copy(
                    ob.at[s],
                    out_hbm.at[0, 0, pl.ds(0, nbw * rows), pl.ds(0, _TW)],
                    osems[s],
                ).wait()

        def compute(s):
            @plsc.parallel_loop(0, nbw * rows * _TW, step=16, unroll=8)
            def gbody(i):
                row = i >> 7
                col = i & (_TW - 1)
                vi = idx_v[s, row, pl.ds(col, 16)]
                o0_v[s, row, pl.ds(col, 16)] = plsc.load_gather(t0_v, [vi])
                o1_v[s, row, pl.ds(col, 16)] = plsc.load_gather(t1_v, [vi])

        start_staging(0)
        pltpu.sync_copy(tableT_hbm.at[d0], t0_v)
        pltpu.sync_copy(tableT_hbm.at[d1], t1_v)
        finish_staging()

        for im in range(b):
            c0 = im * nbh
            if im + 1 < b:
                start_staging(im + 1)
            start_idx(c0, 0)
            start_idx(c0 + 1, 1)

            def body(j, carry, c0=c0):
                ca = c0 + 2 * j
                cb = c0 + 2 * j + 1

                @pl.when(j >= 1)
                def _():
                    wait_out(0)

                wait_idx(0)
                compute(0)
                start_out(ca, 0)

                @pl.when(ca + 2 < c0 + nbh)
                def _():
                    start_idx(ca + 2, 0)

                @pl.when(j >= 1)
                def _():
                    wait_out(1)

                wait_idx(1)
                compute(1)
                start_out(cb, 1)

                @pl.when(cb + 2 < c0 + nbh)
                def _():
                    start_idx(cb + 2, 1)

                return carry

            lax.fori_loop(0, nbh // 2, body, 0)
            wait_out(0)
            wait_out(1)
            if im + 1 < b:
                finish_staging()

    return sc_lookup


def kernel(img, chargrid_map, embedding_table):
    b, _, h, w = chargrid_map.shape
    vocab, d = embedding_table.shape
    vocab_pad = -(-vocab // 2048) * 2048

    table_t = _tc_transpose_table(embedding_table, vocab_pad)
    return _make_sc_lookup(b, d, h, w, vocab_pad)(table_t, chargrid_map)


# speedup vs baseline: 18.5177x; 1.1802x over previous
import functools

import jax
import jax.numpy as jnp
from jax import lax
from jax.experimental import pallas as pl
from jax.experimental.pallas import tpu as pltpu
from jax.experimental.pallas import tpu_sc as plsc

_NC = 2
_NS = 16
_NW = _NC * _NS

_TH = 8
_TW = 128


def _make_sc_lookup(b: int, d: int, h: int, w: int, vocab_pad: int):
    _CH = 2
    nbh = h // (_TH * _CH)
    nbw = w // _TW
    rows = _CH * _TH
    n_chunks = b * nbh
    mesh = plsc.VectorSubcoreMesh(core_axis_name="c", subcore_axis_name="s")

    @functools.partial(
        pl.kernel,
        mesh=mesh,
        compiler_params=pltpu.CompilerParams(
            use_tc_tiling_on_sc=True, needs_layout_passes=False),
        out_type=jax.ShapeDtypeStruct((b, d, h, w), jnp.float32),
        scratch_types=[
            pltpu.VMEM((vocab_pad,), jnp.float32),
            pltpu.VMEM((vocab_pad,), jnp.float32),
            pltpu.VMEM((2, nbw * rows, _TW), jnp.int32),
            pltpu.VMEM((2, nbw * rows, _TW), jnp.float32),
            pltpu.VMEM((2, nbw * rows, _TW), jnp.float32),
            pltpu.VMEM_SHARED((2, h, w), jnp.int32),
            pltpu.SemaphoreType.DMA,
            pltpu.SemaphoreType.DMA,
            pltpu.SemaphoreType.DMA,
            pltpu.SemaphoreType.DMA,
            pltpu.SemaphoreType.DMA,
        ],
    )
    def sc_lookup(tableT_hbm, idx_hbm, out_hbm, t0_v, t1_v,
                  idx_v, o0_v, o1_v, idx_sp, isem0, isem1, osem0, osem1,
                  ssem):
        wid = lax.axis_index("s") * _NC + lax.axis_index("c")
        sid = lax.axis_index("s")
        d0 = wid * 2
        d1 = wid * 2 + 1

        isems = (isem0, isem1)
        osems = (osem0, osem1)

        rows_per_sid = h // _NS

        def start_staging(im):
            pltpu.async_copy(
                idx_hbm.at[im, 0, pl.ds(sid * rows_per_sid, rows_per_sid)],
                idx_sp.at[im % 2, pl.ds(sid * rows_per_sid, rows_per_sid)],
                ssem,
            )

        def finish_staging():
            pltpu.make_async_copy(
                idx_hbm.at[0, 0, pl.ds(0, rows_per_sid)],
                idx_sp.at[0, pl.ds(0, rows_per_sid)],
                ssem,
            ).wait()
            plsc.subcore_barrier()

        def coords(c):
            bi = c // nbh
            hb = c % nbh
            return bi, hb * rows

        def start_idx(c, s):
            bi, h0 = coords(c)
            for wb in range(nbw):
                pltpu.async_copy(
                    idx_sp.at[bi % 2, pl.ds(h0, rows), pl.ds(wb * _TW, _TW)],
                    idx_v.at[s, pl.ds(wb * rows, rows)],
                    isems[s],
                )

        def wait_idx(s):
            pltpu.make_async_copy(
                idx_sp.at[0, pl.ds(0, nbw * rows), pl.ds(0, _TW)],
                idx_v.at[s],
                isems[s],
            ).wait()

        def start_out(c, s):
            bi, h0 = coords(c)
            for dd, ob in ((d0, o0_v), (d1, o1_v)):
                for wb in range(nbw):
                    pltpu.async_copy(
                        ob.at[s, pl.ds(wb * rows, rows)],
                        out_hbm.at[bi, dd, pl.ds(h0, rows),
                                   pl.ds(wb * _TW, _TW)],
                        osems[s],
                    )

        def wait_out(s):
            for ob in (o0_v, o1_v):
                pltpu.make_async_copy(
                    ob.at[s],
                    out_hbm.at[0, 0, pl.ds(0, nbw * rows), pl.ds(0, _TW)],
                    osems[s],
                ).wait()

        def compute(s):
            @plsc.parallel_loop(0, nbw * rows * _TW, step=16, unroll=8)
            def gbody(i):
                row = i >> 7
                col = i & (_TW - 1)
                vi = idx_v[s, row, pl.ds(col, 16)]
                o0_v[s, row, pl.ds(col, 16)] = plsc.load_gather(t0_v, [vi])
                o1_v[s, row, pl.ds(col, 16)] = plsc.load_gather(t1_v, [vi])

        start_staging(0)
        pltpu.sync_copy(tableT_hbm.at[d0], t0_v)
        pltpu.sync_copy(tableT_hbm.at[d1], t1_v)
        finish_staging()

        for im in range(b):
            c0 = im * nbh
            if im + 1 < b:
                start_staging(im + 1)
            start_idx(c0, 0)
            start_idx(c0 + 1, 1)

            def body(j, carry, c0=c0):
                ca = c0 + 2 * j
                cb = c0 + 2 * j + 1

                @pl.when(j >= 1)
                def _():
                    wait_out(0)

                wait_idx(0)
                compute(0)
                start_out(ca, 0)

                @pl.when(ca + 2 < c0 + nbh)
                def _():
                    start_idx(ca + 2, 0)

                @pl.when(j >= 1)
                def _():
                    wait_out(1)

                wait_idx(1)
                compute(1)
                start_out(cb, 1)

                @pl.when(cb + 2 < c0 + nbh)
                def _():
                    start_idx(cb + 2, 1)

                return carry

            lax.fori_loop(0, nbh // 2, body, 0)
            wait_out(0)
            wait_out(1)
            if im + 1 < b:
                finish_staging()

    return sc_lookup


def kernel(img, chargrid_map, embedding_table):
    b, _, h, w = chargrid_map.shape
    vocab, d = embedding_table.shape
    vocab_pad = -(-vocab // 2048) * 2048

    table_pad = jnp.pad(embedding_table, ((0, vocab_pad - vocab), (0, 0)))
    return _make_sc_lookup(b, d, h, w, vocab_pad)(table_pad.T, chargrid_map)


# speedup vs baseline: 18.5507x; 1.0018x over previous
import functools

import jax
import jax.numpy as jnp
from jax import lax
from jax.experimental import pallas as pl
from jax.experimental.pallas import tpu as pltpu
from jax.experimental.pallas import tpu_sc as plsc

_NC = 2
_NS = 16
_NW = _NC * _NS

_TH = 8
_TW = 128


def _make_sc_lookup(b: int, d: int, h: int, w: int, vocab_pad: int):
    _CH = 2
    nbh = h // (_TH * _CH)
    nbw = w // _TW
    rows = _CH * _TH
    n_chunks = b * nbh
    mesh = plsc.VectorSubcoreMesh(core_axis_name="c", subcore_axis_name="s")

    @functools.partial(
        pl.kernel,
        mesh=mesh,
        compiler_params=pltpu.CompilerParams(
            use_tc_tiling_on_sc=True, needs_layout_passes=False),
        out_type=jax.ShapeDtypeStruct((b, d, h, w), jnp.float32),
        scratch_types=[
            pltpu.VMEM((vocab_pad,), jnp.float32),
            pltpu.VMEM((vocab_pad,), jnp.float32),
            pltpu.VMEM((2, nbw * rows, _TW), jnp.int32),
            pltpu.VMEM((2, nbw * rows, _TW), jnp.float32),
            pltpu.VMEM((2, nbw * rows, _TW), jnp.float32),
            pltpu.VMEM_SHARED((2, h, w), jnp.int32),
            pltpu.SemaphoreType.DMA,
            pltpu.SemaphoreType.DMA,
            pltpu.SemaphoreType.DMA,
            pltpu.SemaphoreType.DMA,
            pltpu.SemaphoreType.DMA,
        ],
    )
    def sc_lookup(tableT_hbm, idx_hbm, out_hbm, t0_v, t1_v,
                  idx_v, o0_v, o1_v, idx_sp, isem0, isem1, osem0, osem1,
                  ssem):
        wid = lax.axis_index("s") * _NC + lax.axis_index("c")
        sid = lax.axis_index("s")
        d0 = wid * 2
        d1 = wid * 2 + 1

        isems = (isem0, isem1)
        osems = (osem0, osem1)

        rows_per_sid = h // _NS

        def start_staging(im):
            pltpu.async_copy(
                idx_hbm.at[im, 0, pl.ds(sid * rows_per_sid, rows_per_sid)],
                idx_sp.at[im % 2, pl.ds(sid * rows_per_sid, rows_per_sid)],
                ssem,
            )

        def finish_staging():
            pltpu.make_async_copy(
                idx_hbm.at[0, 0, pl.ds(0, rows_per_sid)],
                idx_sp.at[0, pl.ds(0, rows_per_sid)],
                ssem,
            ).wait()
            plsc.subcore_barrier()

        def coords(c):
            bi = c // nbh
            hb = c % nbh
            return bi, hb * rows

        def start_idx(c, s):
            bi, h0 = coords(c)
            for wb in range(nbw):
                pltpu.async_copy(
                    idx_sp.at[bi % 2, pl.ds(h0, rows), pl.ds(wb * _TW, _TW)],
                    idx_v.at[s, pl.ds(wb * rows, rows)],
                    isems[s],
                )

        def wait_idx(s):
            pltpu.make_async_copy(
                idx_sp.at[0, pl.ds(0, nbw * rows), pl.ds(0, _TW)],
                idx_v.at[s],
                isems[s],
            ).wait()

        def start_out(c, s):
            bi, h0 = coords(c)
            for dd, ob in ((d0, o0_v), (d1, o1_v)):
                for wb in range(nbw):
                    pltpu.async_copy(
                        ob.at[s, pl.ds(wb * rows, rows)],
                        out_hbm.at[bi, dd, pl.ds(h0, rows),
                                   pl.ds(wb * _TW, _TW)],
                        osems[s],
                    )

        def wait_out(s):
            for ob in (o0_v, o1_v):
                pltpu.make_async_copy(
                    ob.at[s],
                    out_hbm.at[0, 0, pl.ds(0, nbw * rows), pl.ds(0, _TW)],
                    osems[s],
                ).wait()

        def compute(s):
            @plsc.parallel_loop(0, nbw * rows * _TW, step=16, unroll=16)
            def gbody(i):
                row = i >> 7
                col = i & (_TW - 1)
                vi = idx_v[s, row, pl.ds(col, 16)]
                o0_v[s, row, pl.ds(col, 16)] = plsc.load_gather(t0_v, [vi])
                o1_v[s, row, pl.ds(col, 16)] = plsc.load_gather(t1_v, [vi])

        start_staging(0)
        pltpu.sync_copy(tableT_hbm.at[d0], t0_v)
        pltpu.sync_copy(tableT_hbm.at[d1], t1_v)
        finish_staging()

        for im in range(b):
            c0 = im * nbh
            if im + 1 < b:
                start_staging(im + 1)
            start_idx(c0, 0)
            start_idx(c0 + 1, 1)

            def body(j, carry, c0=c0):
                ca = c0 + 2 * j
                cb = c0 + 2 * j + 1

                @pl.when(j >= 1)
                def _():
                    wait_out(0)

                wait_idx(0)
                compute(0)
                start_out(ca, 0)

                @pl.when(ca + 2 < c0 + nbh)
                def _():
                    start_idx(ca + 2, 0)

                @pl.when(j >= 1)
                def _():
                    wait_out(1)

                wait_idx(1)
                compute(1)
                start_out(cb, 1)

                @pl.when(cb + 2 < c0 + nbh)
                def _():
                    start_idx(cb + 2, 1)

                return carry

            lax.fori_loop(0, nbh // 2, body, 0)
            wait_out(0)
            wait_out(1)
            if im + 1 < b:
                finish_staging()

    return sc_lookup


def kernel(img, chargrid_map, embedding_table):
    b, _, h, w = chargrid_map.shape
    vocab, d = embedding_table.shape
    vocab_pad = -(-vocab // 2048) * 2048

    table_pad = jnp.pad(embedding_table, ((0, vocab_pad - vocab), (0, 0)))
    return _make_sc_lookup(b, d, h, w, vocab_pad)(table_pad.T, chargrid_map)


# speedup vs baseline: 19.0398x; 1.0264x over previous
import functools

import jax
import jax.numpy as jnp
from jax import lax
from jax.experimental import pallas as pl
from jax.experimental.pallas import tpu as pltpu
from jax.experimental.pallas import tpu_sc as plsc

_NC = 2
_NS = 16
_NW = _NC * _NS

_TH = 8
_TW = 128


def _make_sc_lookup(b: int, d: int, h: int, w: int, vocab_pad: int):
    _CH = 2
    nbh = h // (_TH * _CH)
    nbw = w // _TW
    rows = _CH * _TH
    n_chunks = b * nbh
    mesh = plsc.VectorSubcoreMesh(core_axis_name="c", subcore_axis_name="s")

    @functools.partial(
        pl.kernel,
        mesh=mesh,
        compiler_params=pltpu.CompilerParams(
            use_tc_tiling_on_sc=True, needs_layout_passes=False),
        out_type=jax.ShapeDtypeStruct((b, d, h, w), jnp.float32),
        scratch_types=[
            pltpu.VMEM((vocab_pad,), jnp.float32),
            pltpu.VMEM((vocab_pad,), jnp.float32),
            pltpu.VMEM((2, nbw * rows, _TW), jnp.int32),
            pltpu.VMEM((2, nbw * rows, _TW), jnp.float32),
            pltpu.VMEM((2, nbw * rows, _TW), jnp.float32),
            pltpu.VMEM_SHARED((2, h, w), jnp.int32),
            pltpu.SemaphoreType.DMA,
            pltpu.SemaphoreType.DMA,
            pltpu.SemaphoreType.DMA,
            pltpu.SemaphoreType.DMA,
            pltpu.SemaphoreType.DMA,
        ],
    )
    def sc_lookup(tableT_hbm, idx_hbm, out_hbm, t0_v, t1_v,
                  idx_v, o0_v, o1_v, idx_sp, isem0, isem1, osem0, osem1,
                  ssem):
        wid = lax.axis_index("s") * _NC + lax.axis_index("c")
        sid = lax.axis_index("s")
        d0 = wid * 2
        d1 = wid * 2 + 1

        isems = (isem0, isem1)
        osems = (osem0, osem1)

        rows_per_sid = h // _NS

        def start_staging(im):
            pltpu.async_copy(
                idx_hbm.at[im, 0, pl.ds(sid * rows_per_sid, rows_per_sid)],
                idx_sp.at[im % 2, pl.ds(sid * rows_per_sid, rows_per_sid)],
                ssem,
            )

        def finish_staging():
            pltpu.make_async_copy(
                idx_hbm.at[0, 0, pl.ds(0, rows_per_sid)],
                idx_sp.at[0, pl.ds(0, rows_per_sid)],
                ssem,
            ).wait()
            plsc.subcore_barrier()

        def coords(c):
            bi = c // nbh
            hb = c % nbh
            return bi, hb * rows

        def start_idx(c, s):
            bi, h0 = coords(c)
            for wb in range(nbw):
                pltpu.async_copy(
                    idx_sp.at[bi % 2, pl.ds(h0, rows), pl.ds(wb * _TW, _TW)],
                    idx_v.at[s, pl.ds(wb * rows, rows)],
                    isems[s],
                )

        def wait_idx(s):
            pltpu.make_async_copy(
                idx_sp.at[0, pl.ds(0, nbw * rows), pl.ds(0, _TW)],
                idx_v.at[s],
                isems[s],
            ).wait()

        def start_out(c, s):
            bi, h0 = coords(c)
            for dd, ob in ((d0, o0_v), (d1, o1_v)):
                for wb in range(nbw):
                    pltpu.async_copy(
                        ob.at[s, pl.ds(wb * rows, rows)],
                        out_hbm.at[bi, dd, pl.ds(h0, rows),
                                   pl.ds(wb * _TW, _TW)],
                        osems[s],
                    )

        def wait_out(s):
            for ob in (o0_v, o1_v):
                pltpu.make_async_copy(
                    ob.at[s],
                    out_hbm.at[0, 0, pl.ds(0, nbw * rows), pl.ds(0, _TW)],
                    osems[s],
                ).wait()

        def compute(s):
            @plsc.parallel_loop(0, nbw * rows * _TW, step=16, unroll=16)
            def gbody(i):
                row = i >> 7
                col = i & (_TW - 1)
                vi = idx_v[s, row, pl.ds(col, 16)]
                o0_v[s, row, pl.ds(col, 16)] = plsc.load_gather(t0_v, [vi])
                o1_v[s, row, pl.ds(col, 16)] = plsc.load_gather(t1_v, [vi])

        start_staging(0)
        pltpu.sync_copy(tableT_hbm.at[d0], t0_v)
        pltpu.sync_copy(tableT_hbm.at[d1], t1_v)
        finish_staging()

        for im in range(b):
            c0 = im * nbh
            if im + 1 < b:
                start_staging(im + 1)
            start_idx(c0, 0)
            start_idx(c0 + 1, 1)

            def body(j, carry, c0=c0):
                ca = c0 + 2 * j
                cb = c0 + 2 * j + 1

                @pl.when(j >= 1)
                def _():
                    wait_out(0)

                wait_idx(0)
                compute(0)
                start_out(ca, 0)

                @pl.when(ca + 2 < c0 + nbh)
                def _():
                    start_idx(ca + 2, 0)

                @pl.when(j >= 1)
                def _():
                    wait_out(1)

                wait_idx(1)
                compute(1)
                start_out(cb, 1)

                @pl.when(cb + 2 < c0 + nbh)
                def _():
                    start_idx(cb + 2, 1)

                return carry

            lax.fori_loop(0, nbh // 2, body, 0)
            wait_out(0)
            wait_out(1)
            if im + 1 < b:
                finish_staging()

    return sc_lookup


def kernel(img, chargrid_map, embedding_table):
    b, _, h, w = chargrid_map.shape
    vocab, d = embedding_table.shape
    return _make_sc_lookup(b, d, h, w, vocab)(embedding_table.T, chargrid_map)
